# baseline (speedup 1.0000x reference)
import functools
import math

import jax
import jax.numpy as jnp
import numpy as np
from jax import lax
from jax.experimental import pallas as pl
from jax.experimental.pallas import tpu as pltpu
from jax.experimental.pallas import tpu_sc as plsc

K = 10
A = 1.0
B = 1.0
ALPHA = 0.5

N = 10000
D = 128
E = 320000
NC = 2
NS = 16
H = D // NC
NP = 10240
RPT = NP // NS
RCH = 32
NKCH = RPT // RCH
EPS = E // NS
C = 32
NB = 8
NCH = 632
EPT = NCH * C


def _adjust_ab(a, b):
    if a + b <= -1.0:
        gap = -a - b - 1.0 + 0.0001
        a = a + gap / 2
        b = b + gap / 2
    return a, b


def _jacobi_ABC(n):
    a, b = _adjust_ab(A, B)
    nab = 2 * n + a + b
    denom = 2 * n * (nab - n) * (nab - 2)
    An = nab * (nab - 1) * (nab - 2) / denom
    Bn = (nab - 1) * (a * a - b * b) / denom
    Cn = -2 * (n + a - 1) * (n + b - 1) * nab / denom
    return An, Bn, Cn


def _norm_weights():
    a, b = _adjust_ab(A, B)
    ws = []
    for i in range(K + 1):
        term1 = (2.0 ** (a + b + 1)) / (2 * i + a + b + 1)
        term2 = math.exp(math.lgamma(i + a + 1) - math.lgamma(i + a + b + 1))
        term3 = math.exp(math.lgamma(i + b + 1) - math.lgamma(i + 1))
        ws.append(math.sqrt(term1 * term2 * term3))
    return ws


def _sc_body(abc, row_hbm, col_hbm, x2_hbm, tab_hbm,
             ret_hbm, gp_hbm, gp2_hbm, acc_hbm,
             row_v, col_v, buf0, buf1, buf2, buf3, buf4, buf5, buf6, buf7,
             z64, s_v, gp_v, gp2_v, av_v, gn_v,
             dinv2_v, dsqrt_v, tab_v, acc_sp,
             gsem0, gsem1, gsem2, gsem3, gsem4, gsem5, gsem6, gsem7,
             ssem0, ssem1, ssem2, ssem3, ssem4, ssem5, ssem6, ssem7):
    bufs = (buf0, buf1, buf2, buf3, buf4, buf5, buf6, buf7)
    gsems = (gsem0, gsem1, gsem2, gsem3, gsem4, gsem5, gsem6, gsem7)
    ssems = (ssem0, ssem1, ssem2, ssem3, ssem4, ssem5, ssem6, ssem7)
    c = lax.axis_index("c")
    s = lax.axis_index("s")
    rbase = s * RPT
    gbase = c * NP + rbase

    zv = jnp.zeros((16,), jnp.float32)
    onev = jnp.ones((16,), jnp.float32)

    pltpu.sync_copy(row_hbm.at[s], row_v)
    pltpu.sync_copy(col_hbm.at[s], col_v)
    pltpu.sync_copy(tab_hbm, tab_v)

    def _fill_z64(i, _):
        for f in range(H // 16):
            z64[i, pl.ds(f * 16, 16)] = zv
        return 0
    lax.fori_loop(0, RCH, _fill_z64, 0)

    def _fill_ones(i, _):
        for f in range(H // 16):
            buf0[i, pl.ds(f * 16, 16)] = onev
        return 0
    lax.fori_loop(0, C, _fill_ones, 0)

    bvec = jnp.zeros((16,), jnp.int32) + c * NP

    def _bias(i, _):
        row_v[pl.ds(i * 16, 16)] = row_v[pl.ds(i * 16, 16)] + bvec
        return 0
    lax.fori_loop(0, EPT // 16, _bias, 0)

    def _zeroacc(k, _):
        pltpu.sync_copy(z64, acc_sp.at[pl.ds(rbase + k * RCH, RCH)])
        return 0
    lax.fori_loop(0, NKCH, _zeroacc, 0)
    plsc.subcore_barrier()

    def _degscat(j, _):
        pltpu.sync_copy(buf0, acc_sp.at[col_v.at[j]], add=True)
        return 0
    lax.fori_loop(0, NCH, _degscat, 0)
    plsc.subcore_barrier()

    def _degread(k, _):
        rb = rbase + k * RCH
        pltpu.sync_copy(acc_sp.at[pl.ds(rb, RCH)], s_v)
        pltpu.sync_copy(z64, acc_sp.at[pl.ds(rb, RCH)])

        def _rows(r, _):
            d = s_v[r, pl.ds(0, 16)]
            ii = lax.bitcast_convert_type(d, jnp.int32)
            ii = (jnp.full((16,), 0x5F3759DF, jnp.int32)
                  - lax.shift_right_arithmetic(ii, jnp.ones((16,), jnp.int32)))
            y = lax.bitcast_convert_type(ii, jnp.float32)
            for _unused in range(4):
                y = y * (1.5 - 0.5 * d * y * y)
            y = jnp.where(d > 0.0, y, jnp.zeros((16,), jnp.float32))
            lr = k * RCH + r
            dinv2_v[lr, pl.ds(0, 16)] = y * y
            dsqrt_v[lr, pl.ds(0, 16)] = d * y
            return 0
        lax.fori_loop(0, RCH, _rows, 0)
        return 0
    lax.fori_loop(0, NKCH, _degread, 0)

    w0 = tab_v[0, pl.ds(0, 16)]

    def _init(k, _):
        gb = gbase + k * RCH
        pltpu.sync_copy(x2_hbm.at[pl.ds(gb, RCH)], gp2_v)

        def _rows(r, _):
            lr = k * RCH + r
            di_r = dinv2_v[lr, pl.ds(0, 16)] * dsqrt_v[lr, pl.ds(0, 16)]
            for f in range(H // 16):
                sl = pl.ds(f * 16, 16)
                g0 = di_r * gp2_v[r, sl]
                gn_v[r, sl] = g0
                av_v[r, sl] = w0 * g0
            return 0
        lax.fori_loop(0, RCH, _rows, 0)
        pltpu.sync_copy(gn_v, gp_hbm.at[pl.ds(gb, RCH)])
        pltpu.sync_copy(av_v, acc_hbm.at[pl.ds(gb, RCH)])
        return 0
    lax.fori_loop(0, NKCH, _init, 0)
    plsc.subcore_barrier()

    srcs = (gp_hbm, gp2_hbm)
    G = NCH // NB
    for i in range(1, K + 1):
        src = srcs[(i - 1) % 2]
        dst = srcs[i % 2]

        for b in range(NB):
            pltpu.async_copy(
                src.at[row_v.at[pl.ds(b * C, C)]], bufs[b], gsems[b])

        def _edge(g, _, src=src):
            for b in range(NB):
                j = g * NB + b
                pltpu.make_async_copy(
                    src.at[row_v.at[pl.ds(j * C, C)]], bufs[b],
                    gsems[b]).wait()
                pltpu.async_copy(
                    bufs[b], acc_sp.at[col_v.at[j]], ssems[b], add=True)
            for b in range(NB):
                j = g * NB + b
                pltpu.make_async_copy(
                    bufs[b], acc_sp.at[col_v.at[j]], ssems[b]).wait()

                @pl.when(g < G - 1)
                def _next_gather(b=b, j=j, src=src):
                    pltpu.async_copy(
                        src.at[row_v.at[pl.ds((j + NB) * C, C)]], bufs[b],
                        gsems[b])
            return 0
        lax.fori_loop(0, G, _edge, 0)
        plsc.subcore_barrier()

        An_s = float(abc[0][i])
        Bn_s = float(abc[1][i])
        Cn_s = float(abc[2][i])
        w_s = tab_v[i, pl.ds(0, 16)]

        def _node(k, _, An_s=An_s, Bn_s=Bn_s, Cn_s=Cn_s, w_s=w_s,
                  src=src, dst=dst, first=(i == 1)):
            rb = rbase + k * RCH
            gb = gbase + k * RCH
            d1 = pltpu.async_copy(acc_sp.at[pl.ds(rb, RCH)], s_v, gsem0)
            d2 = pltpu.async_copy(src.at[pl.ds(gb, RCH)], gp_v, gsem1)
            if not first:
                d3 = pltpu.async_copy(dst.at[pl.ds(gb, RCH)], gp2_v, gsem2)
            d4 = pltpu.async_copy(acc_hbm.at[pl.ds(gb, RCH)], av_v, gsem3)
            d1.wait()
            d2.wait()
            if not first:
                d3.wait()
            d4.wait()
            dz = pltpu.async_copy(z64, acc_sp.at[pl.ds(rb, RCH)], ssem0)

            def _rows(r, _):
                lr = k * RCH + r
                ad = An_s * dinv2_v[lr, pl.ds(0, 16)]
                for f in range(H // 16):
                    sl = pl.ds(f * 16, 16)
                    gnew = Bn_s * gp_v[r, sl] + ad * s_v[r, sl]
                    if not first:
                        gnew = gnew + Cn_s * gp2_v[r, sl]
                    gn_v[r, sl] = gnew
                    av_v[r, sl] = av_v[r, sl] + w_s * gnew
                return 0
            lax.fori_loop(0, RCH, _rows, 0)
            w1 = pltpu.async_copy(gn_v, dst.at[pl.ds(gb, RCH)], ssem1)
            w2 = pltpu.async_copy(av_v, acc_hbm.at[pl.ds(gb, RCH)], ssem2)
            dz.wait()
            w1.wait()
            w2.wait()
            return 0
        lax.fori_loop(0, NKCH, _node, 0)
        plsc.subcore_barrier()

    sigma = tab_v[11, pl.ds(0, 16)]
    zero16 = jnp.zeros((16,), jnp.float32)

    def _out(k, _):
        gb = gbase + k * RCH
        pltpu.sync_copy(acc_hbm.at[pl.ds(gb, RCH)], av_v)
        pltpu.sync_copy(x2_hbm.at[pl.ds(gb, RCH)], gp_v)

        def _rows(r, _):
            lr = k * RCH + r
            dsq = dsqrt_v[lr, pl.ds(0, 16)]
            sel = jnp.where(dsq == 0.0, sigma, zero16)
            for f in range(H // 16):
                sl = pl.ds(f * 16, 16)
                gn_v[r, sl] = dsq * av_v[r, sl] + sel * gp_v[r, sl]
            return 0
        lax.fori_loop(0, RCH, _rows, 0)
        pltpu.sync_copy(gn_v, ret_hbm.at[pl.ds(gb, RCH)])
        return 0
    lax.fori_loop(0, NKCH, _out, 0)


@functools.partial(jax.jit, static_argnums=(4,))
def _jacobi_sc(row_p, col3, x2, tab, abc):
    mesh = plsc.VectorSubcoreMesh(
        core_axis_name="c", subcore_axis_name="s",
        num_cores=NC, num_subcores=NS)
    f32 = jnp.float32
    out_type = [jax.ShapeDtypeStruct((2 * NP, H), f32) for _ in range(4)]
    scratch = [
        pltpu.VMEM((EPT,), jnp.int32),
        pltpu.VMEM((NCH, C), jnp.int32),
        pltpu.VMEM((C, H), f32),
        pltpu.VMEM((C, H), f32),
        pltpu.VMEM((C, H), f32),
        pltpu.VMEM((C, H), f32),
        pltpu.VMEM((C, H), f32),
        pltpu.VMEM((C, H), f32),
        pltpu.VMEM((C, H), f32),
        pltpu.VMEM((C, H), f32),
        pltpu.VMEM((RCH, H), f32),
        pltpu.VMEM((RCH, H), f32),
        pltpu.VMEM((RCH, H), f32),
        pltpu.VMEM((RCH, H), f32),
        pltpu.VMEM((RCH, H), f32),
        pltpu.VMEM((RCH, H), f32),
        pltpu.VMEM((RPT, 16), f32),
        pltpu.VMEM((RPT, 16), f32),
        pltpu.VMEM((16, 16), f32),
        pltpu.VMEM_SHARED((NP, H), f32),
        pltpu.SemaphoreType.DMA,
        pltpu.SemaphoreType.DMA,
        pltpu.SemaphoreType.DMA,
        pltpu.SemaphoreType.DMA,
        pltpu.SemaphoreType.DMA,
        pltpu.SemaphoreType.DMA,
        pltpu.SemaphoreType.DMA,
        pltpu.SemaphoreType.DMA,
        pltpu.SemaphoreType.DMA,
        pltpu.SemaphoreType.DMA,
        pltpu.SemaphoreType.DMA,
        pltpu.SemaphoreType.DMA,
        pltpu.SemaphoreType.DMA,
        pltpu.SemaphoreType.DMA,
        pltpu.SemaphoreType.DMA,
        pltpu.SemaphoreType.DMA,
    ]
    fn = pl.kernel(functools.partial(_sc_body, abc),
                   out_type=out_type, mesh=mesh, scratch_types=scratch,
                   compiler_params=pltpu.CompilerParams(
                       use_tc_tiling_on_sc=False))
    ret, _, _, _ = fn(row_p, col3, x2, tab)
    return ret


def kernel(x, edge_index, lap_coefs, mf_weights):
    a, b = _adjust_ab(A, B)
    c0 = (a - b) / 2.0
    c1 = (a + b + 2.0) / 2.0

    An = np.zeros(16, np.float32)
    Bn = np.zeros(16, np.float32)
    Cn = np.zeros(16, np.float32)
    An[1], Bn[1], Cn[1] = c1, c0, 0.0
    p = np.zeros(K + 1, np.float64)
    p[0], p[1] = 1.0, c0
    for i in range(2, K + 1):
        ai, bi, ci = _jacobi_ABC(i)
        An[i], Bn[i], Cn[i] = ai, bi, ci
        p[i] = bi * p[i - 1] + ci * p[i - 2]

    nw = np.asarray(_norm_weights(), np.float64)
    lap = jnp.cumprod(ALPHA * jnp.tanh(lap_coefs.astype(jnp.float32)))
    mfw = mf_weights.reshape(K + 1).astype(jnp.float32)
    w = jnp.concatenate([
        (mfw[:1] / nw[0]).astype(jnp.float32),
        (mfw[1:] * lap[:K] / nw[1:].astype(np.float32)),
    ])
    sigma = jnp.dot(w, jnp.asarray(p, jnp.float32))

    vals = jnp.concatenate([w, sigma[None],
                            jnp.zeros((16 - (K + 2),), jnp.float32)])
    tab = jnp.tile(vals[:, None], (1, 16))
    abc = (tuple(An.tolist()), tuple(Bn.tolist()), tuple(Cn.tolist()))

    row2 = edge_index[0].reshape(NS, EPS)
    col2 = edge_index[1].reshape(NS, EPS)
    row_p = jnp.pad(row2, ((0, 0), (0, EPT - EPS)))
    col3 = jnp.pad(col2, ((0, 0), (0, EPT - EPS)),
                   constant_values=N).reshape(NS, NCH, C)

    x_pad = jnp.pad(x, ((0, NP - N), (0, 0)))
    x2 = jnp.concatenate([x_pad[:, :H], x_pad[:, H:]], axis=0)

    ret = _jacobi_sc(row_p, col3, x2, tab, abc)
    return jnp.concatenate([ret[:N], ret[NP:NP + N]], axis=1)


# speedup vs baseline: 1.0518x; 1.0518x over previous
import functools
import math

import jax
import jax.numpy as jnp
import numpy as np
from jax import lax
from jax.experimental import pallas as pl
from jax.experimental.pallas import tpu as pltpu
from jax.experimental.pallas import tpu_sc as plsc

K = 10
A = 1.0
B = 1.0
ALPHA = 0.5

N = 10000
D = 128
E = 320000
NC = 2
NS = 16
H = D // NC
NP = 10240
RPT = NP // NS
RCH = 64
NKCH = RPT // RCH
EPS = E // NS
C = 64
NB = 4
NCH = 316
EPT = NCH * C


def _adjust_ab(a, b):
    if a + b <= -1.0:
        gap = -a - b - 1.0 + 0.0001
        a = a + gap / 2
        b = b + gap / 2
    return a, b


def _jacobi_ABC(n):
    a, b = _adjust_ab(A, B)
    nab = 2 * n + a + b
    denom = 2 * n * (nab - n) * (nab - 2)
    An = nab * (nab - 1) * (nab - 2) / denom
    Bn = (nab - 1) * (a * a - b * b) / denom
    Cn = -2 * (n + a - 1) * (n + b - 1) * nab / denom
    return An, Bn, Cn


def _norm_weights():
    a, b = _adjust_ab(A, B)
    ws = []
    for i in range(K + 1):
        term1 = (2.0 ** (a + b + 1)) / (2 * i + a + b + 1)
        term2 = math.exp(math.lgamma(i + a + 1) - math.lgamma(i + a + b + 1))
        term3 = math.exp(math.lgamma(i + b + 1) - math.lgamma(i + 1))
        ws.append(math.sqrt(term1 * term2 * term3))
    return ws


def _sc_body(abc, row_hbm, col_hbm, x2_hbm, tab_hbm,
             ret_hbm, gp_hbm, gp2_hbm, acc_hbm,
             row_v, col_v, buf0, buf1, buf2, buf3, z64,
             s_v, gp_v, gp2_v, av_v,
             di_v, tab_v, acc_sp,
             gsem0, gsem1, gsem2, gsem3, ssem0, ssem1, ssem2, ssem3):
    bufs = (buf0, buf1, buf2, buf3)
    gsems = (gsem0, gsem1, gsem2, gsem3)
    ssems = (ssem0, ssem1, ssem2, ssem3)
    c = lax.axis_index("c")
    s = lax.axis_index("s")
    rbase = s * RPT
    gbase = c * NP + rbase

    zv = jnp.zeros((16,), jnp.float32)
    onev = jnp.ones((16,), jnp.float32)

    pltpu.sync_copy(row_hbm.at[s], row_v)
    pltpu.sync_copy(col_hbm.at[s], col_v)
    pltpu.sync_copy(tab_hbm, tab_v)

    def _fill_z64(i, _):
        for f in range(H // 16):
            z64[i, pl.ds(f * 16, 16)] = zv
        return 0
    lax.fori_loop(0, RCH, _fill_z64, 0)

    def _fill_ones(i, _):
        for f in range(H // 16):
            buf0[i, pl.ds(f * 16, 16)] = onev
        return 0
    lax.fori_loop(0, C, _fill_ones, 0)

    bvec = jnp.zeros((16,), jnp.int32) + c * NP

    def _bias(i, _):
        row_v[pl.ds(i * 16, 16)] = row_v[pl.ds(i * 16, 16)] + bvec
        return 0
    lax.fori_loop(0, EPT // 16, _bias, 0)

    def _zeroacc(k, _):
        pltpu.sync_copy(z64, acc_sp.at[pl.ds(rbase + k * RCH, RCH)])
        return 0
    lax.fori_loop(0, NKCH, _zeroacc, 0)
    plsc.subcore_barrier()

    def _degscat(j, _):
        pltpu.sync_copy(buf0, acc_sp.at[col_v.at[j]], add=True)
        return 0
    lax.fori_loop(0, NCH, _degscat, 0)
    plsc.subcore_barrier()

    def _degread(k, _):
        rb = rbase + k * RCH
        pltpu.sync_copy(acc_sp.at[pl.ds(rb, RCH)], s_v)
        pltpu.sync_copy(z64, acc_sp.at[pl.ds(rb, RCH)])

        def _rows(r, _):
            d = s_v[r, pl.ds(0, 16)]
            ii = lax.bitcast_convert_type(d, jnp.int32)
            ii = (jnp.full((16,), 0x5F3759DF, jnp.int32)
                  - lax.shift_right_arithmetic(ii, jnp.ones((16,), jnp.int32)))
            y = lax.bitcast_convert_type(ii, jnp.float32)
            for _unused in range(4):
                y = y * (1.5 - 0.5 * d * y * y)
            y = jnp.where(d > 0.0, y, jnp.zeros((16,), jnp.float32))
            lr = k * RCH + r
            di_v[lr, pl.ds(0, 16)] = y
            return 0
        lax.fori_loop(0, RCH, _rows, 0)
        return 0
    lax.fori_loop(0, NKCH, _degread, 0)

    w0 = tab_v[0, pl.ds(0, 16)]

    def _init(k, _):
        gb = gbase + k * RCH
        pltpu.sync_copy(x2_hbm.at[pl.ds(gb, RCH)], gp2_v)

        def _rows(r, _):
            lr = k * RCH + r
            di_r = di_v[lr, pl.ds(0, 16)]
            for f in range(H // 16):
                sl = pl.ds(f * 16, 16)
                g0 = di_r * gp2_v[r, sl]
                s_v[r, sl] = g0
                av_v[r, sl] = w0 * g0
            return 0
        lax.fori_loop(0, RCH, _rows, 0)
        pltpu.sync_copy(s_v, gp_hbm.at[pl.ds(gb, RCH)])
        pltpu.sync_copy(av_v, acc_hbm.at[pl.ds(gb, RCH)])
        return 0
    lax.fori_loop(0, NKCH, _init, 0)
    plsc.subcore_barrier()

    srcs = (gp_hbm, gp2_hbm)
    G = NCH // NB
    for i in range(1, K + 1):
        src = srcs[(i - 1) % 2]
        dst = srcs[i % 2]

        for b in range(NB):
            pltpu.async_copy(
                src.at[row_v.at[pl.ds(b * C, C)]], bufs[b], gsems[b])

        def _edge(g, _, src=src):
            for b in range(NB):
                j = g * NB + b
                pltpu.make_async_copy(
                    src.at[row_v.at[pl.ds(j * C, C)]], bufs[b],
                    gsems[b]).wait()
                pltpu.async_copy(
                    bufs[b], acc_sp.at[col_v.at[j]], ssems[b], add=True)
            for b in range(NB):
                j = g * NB + b
                pltpu.make_async_copy(
                    bufs[b], acc_sp.at[col_v.at[j]], ssems[b]).wait()

                @pl.when(g < G - 1)
                def _next_gather(b=b, j=j, src=src):
                    pltpu.async_copy(
                        src.at[row_v.at[pl.ds((j + NB) * C, C)]], bufs[b],
                        gsems[b])
            return 0
        lax.fori_loop(0, G, _edge, 0)
        plsc.subcore_barrier()

        An_s = float(abc[0][i])
        Bn_s = float(abc[1][i])
        Cn_s = float(abc[2][i])
        w_s = tab_v[i, pl.ds(0, 16)]

        def _node(k, _, An_s=An_s, Bn_s=Bn_s, Cn_s=Cn_s, w_s=w_s,
                  src=src, dst=dst, first=(i == 1)):
            rb = rbase + k * RCH
            gb = gbase + k * RCH
            d1 = pltpu.async_copy(acc_sp.at[pl.ds(rb, RCH)], s_v, gsem0)
            d2 = pltpu.async_copy(src.at[pl.ds(gb, RCH)], gp_v, gsem1)
            if not first:
                d3 = pltpu.async_copy(dst.at[pl.ds(gb, RCH)], gp2_v, gsem2)
            d4 = pltpu.async_copy(acc_hbm.at[pl.ds(gb, RCH)], av_v, gsem3)
            d1.wait()
            d2.wait()
            if not first:
                d3.wait()
            d4.wait()
            dz = pltpu.async_copy(z64, acc_sp.at[pl.ds(rb, RCH)], ssem0)

            def _rows(r, _):
                lr = k * RCH + r
                dd = di_v[lr, pl.ds(0, 16)]
                ad = (An_s * dd) * dd
                for f in range(H // 16):
                    sl = pl.ds(f * 16, 16)
                    gnew = Bn_s * gp_v[r, sl] + ad * s_v[r, sl]
                    if not first:
                        gnew = gnew + Cn_s * gp2_v[r, sl]
                    s_v[r, sl] = gnew
                    av_v[r, sl] = av_v[r, sl] + w_s * gnew
                return 0
            lax.fori_loop(0, RCH, _rows, 0)
            w1 = pltpu.async_copy(s_v, dst.at[pl.ds(gb, RCH)], ssem1)
            w2 = pltpu.async_copy(av_v, acc_hbm.at[pl.ds(gb, RCH)], ssem2)
            dz.wait()
            w1.wait()
            w2.wait()
            return 0
        lax.fori_loop(0, NKCH, _node, 0)
        plsc.subcore_barrier()

    sigma = tab_v[11, pl.ds(0, 16)]
    zero16 = jnp.zeros((16,), jnp.float32)

    def _out(k, _):
        gb = gbase + k * RCH
        pltpu.sync_copy(acc_hbm.at[pl.ds(gb, RCH)], av_v)
        pltpu.sync_copy(x2_hbm.at[pl.ds(gb, RCH)], gp_v)

        def _rows(r, _):
            lr = k * RCH + r
            dd = di_v[lr, pl.ds(0, 16)]
            dsq = jnp.where(dd > 0.0, 1.0 / dd, zero16)
            sel = jnp.where(dd == 0.0, sigma, zero16)
            for f in range(H // 16):
                sl = pl.ds(f * 16, 16)
                s_v[r, sl] = dsq * av_v[r, sl] + sel * gp_v[r, sl]
            return 0
        lax.fori_loop(0, RCH, _rows, 0)
        pltpu.sync_copy(s_v, ret_hbm.at[pl.ds(gb, RCH)])
        return 0
    lax.fori_loop(0, NKCH, _out, 0)


@functools.partial(jax.jit, static_argnums=(4,))
def _jacobi_sc(row_p, col3, x2, tab, abc):
    mesh = plsc.VectorSubcoreMesh(
        core_axis_name="c", subcore_axis_name="s",
        num_cores=NC, num_subcores=NS)
    f32 = jnp.float32
    out_type = [jax.ShapeDtypeStruct((2 * NP, H), f32) for _ in range(4)]
    scratch = [
        pltpu.VMEM((EPT,), jnp.int32),
        pltpu.VMEM((NCH, C), jnp.int32),
        pltpu.VMEM((C, H), f32),
        pltpu.VMEM((C, H), f32),
        pltpu.VMEM((C, H), f32),
        pltpu.VMEM((C, H), f32),
        pltpu.VMEM((RCH, H), f32),
        pltpu.VMEM((RCH, H), f32),
        pltpu.VMEM((RCH, H), f32),
        pltpu.VMEM((RCH, H), f32),
        pltpu.VMEM((RCH, H), f32),
        pltpu.VMEM((RPT, 16), f32),
        pltpu.VMEM((16, 16), f32),
        pltpu.VMEM_SHARED((NP, H), f32),
        pltpu.SemaphoreType.DMA,
        pltpu.SemaphoreType.DMA,
        pltpu.SemaphoreType.DMA,
        pltpu.SemaphoreType.DMA,
        pltpu.SemaphoreType.DMA,
        pltpu.SemaphoreType.DMA,
        pltpu.SemaphoreType.DMA,
        pltpu.SemaphoreType.DMA,
    ]
    fn = pl.kernel(functools.partial(_sc_body, abc),
                   out_type=out_type, mesh=mesh, scratch_types=scratch,
                   compiler_params=pltpu.CompilerParams(
                       use_tc_tiling_on_sc=False))
    ret, _, _, _ = fn(row_p, col3, x2, tab)
    return ret


def kernel(x, edge_index, lap_coefs, mf_weights):
    a, b = _adjust_ab(A, B)
    c0 = (a - b) / 2.0
    c1 = (a + b + 2.0) / 2.0

    An = np.zeros(16, np.float32)
    Bn = np.zeros(16, np.float32)
    Cn = np.zeros(16, np.float32)
    An[1], Bn[1], Cn[1] = c1, c0, 0.0
    p = np.zeros(K + 1, np.float64)
    p[0], p[1] = 1.0, c0
    for i in range(2, K + 1):
        ai, bi, ci = _jacobi_ABC(i)
        An[i], Bn[i], Cn[i] = ai, bi, ci
        p[i] = bi * p[i - 1] + ci * p[i - 2]

    nw = np.asarray(_norm_weights(), np.float64)
    lap = jnp.cumprod(ALPHA * jnp.tanh(lap_coefs.astype(jnp.float32)))
    mfw = mf_weights.reshape(K + 1).astype(jnp.float32)
    w = jnp.concatenate([
        (mfw[:1] / nw[0]).astype(jnp.float32),
        (mfw[1:] * lap[:K] / nw[1:].astype(np.float32)),
    ])
    sigma = jnp.dot(w, jnp.asarray(p, jnp.float32))

    vals = jnp.concatenate([w, sigma[None],
                            jnp.zeros((16 - (K + 2),), jnp.float32)])
    tab = jnp.tile(vals[:, None], (1, 16))
    abc = (tuple(An.tolist()), tuple(Bn.tolist()), tuple(Cn.tolist()))

    row2 = edge_index[0].reshape(NS, EPS)
    col2 = edge_index[1].reshape(NS, EPS)
    row_p = jnp.pad(row2, ((0, 0), (0, EPT - EPS)))
    col3 = jnp.pad(col2, ((0, 0), (0, EPT - EPS)),
                   constant_values=N).reshape(NS, NCH, C)

    x_pad = jnp.pad(x, ((0, NP - N), (0, 0)))
    x2 = jnp.concatenate([x_pad[:, :H], x_pad[:, H:]], axis=0)

    ret = _jacobi_sc(row_p, col3, x2, tab, abc)
    return jnp.concatenate([ret[:N], ret[NP:NP + N]], axis=1)
